# baseline (speedup 1.0000x reference)
import functools

import jax
import jax.numpy as jnp
from jax.experimental import pallas as pl
from jax.experimental.pallas import tpu as pltpu

EMBED = 2048
TILE_V = 1024


def _proj_kernel(x_ref, w_ref, b_ref, o_ref):
    acc = jax.lax.dot_general(
        x_ref[:, :], w_ref[:, :],
        dimension_numbers=(((1,), (1,)), ((), ())),
        preferred_element_type=jnp.float32,
    )
    o_ref[:, :] = acc + b_ref[:, :]


def kernel(x, W, b):
    batch, embed = x.shape
    vocab = W.shape[0]
    b2 = b.reshape(1, vocab)
    grid = (pl.cdiv(vocab, TILE_V),)
    return pl.pallas_call(
        _proj_kernel,
        grid=grid,
        in_specs=[
            pl.BlockSpec((batch, embed), lambda i: (0, 0)),
            pl.BlockSpec((TILE_V, embed), lambda i: (i, 0)),
            pl.BlockSpec((1, TILE_V), lambda i: (0, i)),
        ],
        out_specs=pl.BlockSpec((batch, TILE_V), lambda i: (0, i)),
        out_shape=jax.ShapeDtypeStruct((batch, vocab), jnp.float32),
        compiler_params=pltpu.CompilerParams(
            dimension_semantics=("parallel",),
        ),
    )(x, W, b2)


# speedup vs baseline: 1.0083x; 1.0083x over previous
import functools

import jax
import jax.numpy as jnp
from jax.experimental import pallas as pl
from jax.experimental.pallas import tpu as pltpu

EMBED = 2048
TILE_V = 3072


def _proj_kernel(x_ref, w_ref, b_ref, o_ref):
    acc = jax.lax.dot_general(
        x_ref[:, :], w_ref[:, :],
        dimension_numbers=(((1,), (1,)), ((), ())),
        preferred_element_type=jnp.float32,
    )
    o_ref[:, :] = acc + b_ref[:, :]


def kernel(x, W, b):
    batch, embed = x.shape
    vocab = W.shape[0]
    b2 = b.reshape(1, vocab)
    grid = (pl.cdiv(vocab, TILE_V),)
    return pl.pallas_call(
        _proj_kernel,
        grid=grid,
        in_specs=[
            pl.BlockSpec((batch, embed), lambda i: (0, 0)),
            pl.BlockSpec((TILE_V, embed), lambda i: (i, 0)),
            pl.BlockSpec((1, TILE_V), lambda i: (0, i)),
        ],
        out_specs=pl.BlockSpec((batch, TILE_V), lambda i: (0, i)),
        out_shape=jax.ShapeDtypeStruct((batch, vocab), jnp.float32),
        compiler_params=pltpu.CompilerParams(
            dimension_semantics=("parallel",),
        ),
    )(x, W, b2)


# speedup vs baseline: 1.0158x; 1.0074x over previous
import functools

import jax
import jax.numpy as jnp
from jax.experimental import pallas as pl
from jax.experimental.pallas import tpu as pltpu

EMBED = 2048
TILE_V = 2048


def _proj_kernel(x_ref, w_ref, b_ref, o_ref):
    acc = jax.lax.dot_general(
        x_ref[:, :], w_ref[:, :],
        dimension_numbers=(((1,), (1,)), ((), ())),
        preferred_element_type=jnp.float32,
    )
    o_ref[:, :] = acc + b_ref[:, :]


def kernel(x, W, b):
    batch, embed = x.shape
    vocab = W.shape[0]
    b2 = b.reshape(1, vocab)
    grid = (pl.cdiv(vocab, TILE_V),)
    return pl.pallas_call(
        _proj_kernel,
        grid=grid,
        in_specs=[
            pl.BlockSpec((batch, embed), lambda i: (0, 0)),
            pl.BlockSpec((TILE_V, embed), lambda i: (i, 0)),
            pl.BlockSpec((1, TILE_V), lambda i: (0, i)),
        ],
        out_specs=pl.BlockSpec((batch, TILE_V), lambda i: (0, i)),
        out_shape=jax.ShapeDtypeStruct((batch, vocab), jnp.float32),
        compiler_params=pltpu.CompilerParams(
            dimension_semantics=("parallel",),
        ),
    )(x, W, b2)


# speedup vs baseline: 1.0164x; 1.0007x over previous
import jax
import jax.numpy as jnp
from jax.experimental import pallas as pl
from jax.experimental.pallas import tpu as pltpu

TILE_V = 2048


def _proj_kernel(x_ref, w_ref, b_ref, o_ref):
    i = pl.program_id(0)
    acc = jax.lax.dot_general(
        x_ref[:, :], w_ref[:, :],
        dimension_numbers=(((1,), (1,)), ((), ())),
        preferred_element_type=jnp.float32,
    )
    o_ref[:, :] = acc + b_ref[:, pl.ds(i * TILE_V, TILE_V)]


def kernel(x, W, b):
    batch, embed = x.shape
    vocab = W.shape[0]
    num_tiles = pl.cdiv(vocab, TILE_V)
    vpad = num_tiles * TILE_V
    b2 = jnp.pad(b, (0, vpad - vocab)).reshape(1, vpad)
    return pl.pallas_call(
        _proj_kernel,
        grid=(num_tiles,),
        in_specs=[
            pl.BlockSpec((batch, embed), lambda i: (0, 0)),
            pl.BlockSpec((TILE_V, embed), lambda i: (i, 0)),
            pl.BlockSpec((1, vpad), lambda i: (0, 0)),
        ],
        out_specs=pl.BlockSpec((batch, TILE_V), lambda i: (0, i)),
        out_shape=jax.ShapeDtypeStruct((batch, vocab), jnp.float32),
        compiler_params=pltpu.CompilerParams(
            dimension_semantics=("arbitrary",),
        ),
    )(x, W, b2)
